# baseline (speedup 1.0000x reference)
import functools
import math

import jax
import jax.numpy as jnp
from jax import lax
from jax.experimental import pallas as pl
from jax.experimental.pallas import tpu as pltpu
from jax.experimental.pallas import tpu_sc as plsc

_S = 64.0
_M2 = 0.5
_COS_M = math.cos(_M2)
_SIN_M = math.sin(_M2)
_THETA = math.cos(math.pi - _M2)
_SINMM = math.sin(math.pi - _M2) * _M2

_BR = 16


def _merge_body(lab_ref, x_ref, o_ref):
    x = x_ref[...]
    lab = lab_ref[...]
    cols = lax.broadcasted_iota(jnp.int32, x.shape, 1)
    mask = cols == lab
    t = jnp.sum(jnp.where(mask, x, 0.0), axis=1, keepdims=True)
    sin_t = jnp.sqrt(1.0 - t * t)
    cos_theta_m = t * _COS_M - sin_t * _SIN_M
    f = jnp.where(t > _THETA, cos_theta_m, t - _SINMM)
    upd = jnp.where(lab >= 0, f, t)
    o_ref[...] = _S * jnp.where(mask, upd, x)


def kernel(logits, labels):
    b, v = logits.shape
    return pl.pallas_call(
        _merge_body,
        grid=(b // _BR,),
        in_specs=[
            pl.BlockSpec((_BR, 1), lambda i: (i, 0)),
            pl.BlockSpec((_BR, v), lambda i: (i, 0)),
        ],
        out_specs=pl.BlockSpec((_BR, v), lambda i: (i, 0)),
        out_shape=jax.ShapeDtypeStruct((b, v), jnp.float32),
    )(labels.reshape(b, 1), logits)
